# baseline (speedup 1.0000x reference)
import functools

import jax
import jax.numpy as jnp
from jax import lax
from jax.experimental import pallas as pl

_NN = (((1,), (1,)), ((), ()))
_NT = (((1,), (0,)), ((), ()))


def _split_dot_nn(a, b):
    ah = a.astype(jnp.bfloat16).astype(jnp.float32)
    al = a - ah
    bh = b.astype(jnp.bfloat16).astype(jnp.float32)
    bl = b - bh
    out = lax.dot_general(a, bl, _NN, preferred_element_type=jnp.float32)
    out += lax.dot_general(al, bh, _NN, preferred_element_type=jnp.float32)
    out += lax.dot_general(ah, bh, _NN, preferred_element_type=jnp.float32)
    return out


def _fp_kernel(uxyz_ref, kxyzt_ref, ufeat_ref, kfeat_ref,
               w0_ref, b0_ref, w1_ref, b1_ref, out_ref, *, m):
    u = uxyz_ref[0]
    kt = kxyzt_ref[0]
    tn = u.shape[0]

    u0, u1, u2 = u[:, 0:1], u[:, 1:2], u[:, 2:3]
    k0, k1, k2 = kt[0:1, :], kt[1:2, :], kt[2:3, :]
    uu = u0 * u0 + u1 * u1 + u2 * u2
    kk = k0 * k0 + k1 * k1 + k2 * k2
    uk = lax.dot_general(u, kt, _NT, preferred_element_type=jnp.float32)
    d2 = (uu + kk) - 2.0 * uk

    iota = lax.broadcasted_iota(jnp.int32, (tn, m), 1).astype(jnp.float32)
    cur = d2
    idxs, rs = [], []
    for j in range(3):
        mn = jnp.min(cur, axis=1, keepdims=True)
        idx = jnp.min(jnp.where(cur == mn, iota, jnp.float32(m)),
                      axis=1, keepdims=True)
        idxs.append(idx)
        rs.append(1.0 / (jnp.maximum(mn, 0.0) + 1e-8))
        if j < 2:
            cur = jnp.where(iota == idx, jnp.float32(jnp.inf), cur)
    inv = 1.0 / ((rs[0] + rs[1]) + rs[2])
    w0_, w1_, w2_ = rs[0] * inv, rs[1] * inv, rs[2] * inv
    wmat = jnp.where(iota == idxs[0], w0_,
                     jnp.where(iota == idxs[1], w1_,
                               jnp.where(iota == idxs[2], w2_, 0.0)))

    interp = _split_dot_nn(kfeat_ref[0], wmat)

    x = jnp.concatenate([interp, ufeat_ref[0]], axis=0)
    h = lax.dot_general(w0_ref[:], x, _NT, preferred_element_type=jnp.float32)
    h = jnp.maximum(h + b0_ref[:], 0.0)
    o = lax.dot_general(w1_ref[:], h, _NT, preferred_element_type=jnp.float32)
    out_ref[0] = jnp.maximum(o + b1_ref[:], 0.0)


def _fp_level(uxyz, kxyz, ufeat, kfeat, w0, b0, w1, b1, tn, interpret=False):
    B, n, _ = uxyz.shape
    m = kxyz.shape[1]
    cu = ufeat.shape[1]
    ck = kfeat.shape[1]
    o, cin = w0.shape
    grid = (B, n // tn)
    kxyzt = jnp.transpose(kxyz, (0, 2, 1))
    return pl.pallas_call(
        functools.partial(_fp_kernel, m=m),
        grid=grid,
        in_specs=[
            pl.BlockSpec((1, tn, 3), lambda b, t: (b, t, 0)),
            pl.BlockSpec((1, 3, m), lambda b, t: (b, 0, 0)),
            pl.BlockSpec((1, cu, tn), lambda b, t: (b, 0, t)),
            pl.BlockSpec((1, ck, m), lambda b, t: (b, 0, 0)),
            pl.BlockSpec((o, cin), lambda b, t: (0, 0)),
            pl.BlockSpec((o, 1), lambda b, t: (0, 0)),
            pl.BlockSpec((o, o), lambda b, t: (0, 0)),
            pl.BlockSpec((o, 1), lambda b, t: (0, 0)),
        ],
        out_specs=pl.BlockSpec((1, o, tn), lambda b, t: (b, 0, t)),
        out_shape=jax.ShapeDtypeStruct((B, o, n), jnp.float32),
        interpret=interpret,
    )(uxyz, kxyzt, ufeat, kfeat, w0, b0.reshape(o, 1), w1, b1.reshape(o, 1))


def kernel(l_xyz_0, l_xyz_1, l_xyz_2, l_xyz_3, l_xyz_4,
           l_features_0, l_features_1, l_features_2, l_features_3, l_features_4,
           fp4_w0, fp4_b0, fp4_w1, fp4_b1,
           fp3_w0, fp3_b0, fp3_w1, fp3_b1,
           fp2_w0, fp2_b0, fp2_w1, fp2_b1,
           fp1_w0, fp1_b0, fp1_w1, fp1_b1):
    f3 = _fp_level(l_xyz_3, l_xyz_4, l_features_3, l_features_4,
                   fp4_w0, fp4_b0, fp4_w1, fp4_b1, tn=64)
    f2 = _fp_level(l_xyz_2, l_xyz_3, l_features_2, f3,
                   fp3_w0, fp3_b0, fp3_w1, fp3_b1, tn=256)
    f1 = _fp_level(l_xyz_1, l_xyz_2, l_features_1, f2,
                   fp2_w0, fp2_b0, fp2_w1, fp2_b1, tn=1024)
    f0 = _fp_level(l_xyz_0, l_xyz_1, l_features_0, f1,
                   fp1_w0, fp1_b0, fp1_w1, fp1_b1, tn=4096)
    return f0


# speedup vs baseline: 1.0582x; 1.0582x over previous
import functools

import jax
import jax.numpy as jnp
from jax import lax
from jax.experimental import pallas as pl

_NN = (((1,), (1,)), ((), ()))
_NT = (((1,), (0,)), ((), ()))


def _split_dot_nn(a, b):
    ah = a.astype(jnp.bfloat16).astype(jnp.float32)
    al = a - ah
    bh = b.astype(jnp.bfloat16).astype(jnp.float32)
    bl = b - bh
    out = lax.dot_general(a, bl, _NN, preferred_element_type=jnp.float32)
    out += lax.dot_general(al, bh, _NN, preferred_element_type=jnp.float32)
    out += lax.dot_general(ah, bh, _NN, preferred_element_type=jnp.float32)
    return out


def _fp_kernel(uxyz_ref, kxyzt_ref, ufeat_ref, kfeat_ref,
               w0_ref, b0_ref, w1_ref, b1_ref, out_ref, *, m, bs):
    if bs == 1:
        u = uxyz_ref[0]
        kt = kxyzt_ref[0]
        uf = ufeat_ref[0]
        kf = kfeat_ref[0]
    else:
        u = jnp.concatenate([uxyz_ref[i] for i in range(bs)], axis=0)
        kt = jnp.concatenate([kxyzt_ref[i] for i in range(bs)], axis=1)
        uf = jnp.concatenate([ufeat_ref[i] for i in range(bs)], axis=1)
        kf = jnp.concatenate([kfeat_ref[i] for i in range(bs)], axis=1)
    tn = u.shape[0]
    sub_m = m
    m = m * bs

    u0, u1, u2 = u[:, 0:1], u[:, 1:2], u[:, 2:3]
    k0, k1, k2 = kt[0:1, :], kt[1:2, :], kt[2:3, :]
    uu = u0 * u0 + u1 * u1 + u2 * u2
    kk = k0 * k0 + k1 * k1 + k2 * k2
    uk = lax.dot_general(u, kt, _NT, preferred_element_type=jnp.float32)
    d2 = (uu + kk) - 2.0 * uk
    if bs > 1:
        br = lax.broadcasted_iota(jnp.int32, (tn, m), 0) // (tn // bs)
        bc = lax.broadcasted_iota(jnp.int32, (tn, m), 1) // sub_m
        d2 = jnp.where(br == bc, d2, jnp.float32(jnp.inf))

    iota = lax.broadcasted_iota(jnp.int32, (tn, m), 1).astype(jnp.float32)
    cur = d2
    idxs, rs = [], []
    for j in range(3):
        mn = jnp.min(cur, axis=1, keepdims=True)
        idx = jnp.min(jnp.where(cur == mn, iota, jnp.float32(m)),
                      axis=1, keepdims=True)
        idxs.append(idx)
        rs.append(1.0 / (jnp.maximum(mn, 0.0) + 1e-8))
        if j < 2:
            cur = jnp.where(iota == idx, jnp.float32(jnp.inf), cur)
    inv = 1.0 / ((rs[0] + rs[1]) + rs[2])
    w0_, w1_, w2_ = rs[0] * inv, rs[1] * inv, rs[2] * inv
    wmat = jnp.where(iota == idxs[0], w0_,
                     jnp.where(iota == idxs[1], w1_,
                               jnp.where(iota == idxs[2], w2_, 0.0)))

    interp = _split_dot_nn(kf, wmat)

    x = jnp.concatenate([interp, uf], axis=0)
    h = lax.dot_general(w0_ref[:], x, _NT, preferred_element_type=jnp.float32)
    h = jnp.maximum(h + b0_ref[:], 0.0)
    o = lax.dot_general(w1_ref[:], h, _NT, preferred_element_type=jnp.float32)
    res = jnp.maximum(o + b1_ref[:], 0.0)
    if bs == 1:
        out_ref[0] = res
    else:
        sub_n = tn // bs
        for i in range(bs):
            out_ref[i] = res[:, i * sub_n:(i + 1) * sub_n]


def _fp_level(uxyz, kxyz, ufeat, kfeat, w0, b0, w1, b1, tn, bs=1,
              interpret=False):
    B, n, _ = uxyz.shape
    m = kxyz.shape[1]
    cu = ufeat.shape[1]
    ck = kfeat.shape[1]
    o, cin = w0.shape
    grid = (B // bs, n // tn)
    kxyzt = jnp.transpose(kxyz, (0, 2, 1))
    return pl.pallas_call(
        functools.partial(_fp_kernel, m=m, bs=bs),
        grid=grid,
        in_specs=[
            pl.BlockSpec((bs, tn, 3), lambda b, t: (b, t, 0)),
            pl.BlockSpec((bs, 3, m), lambda b, t: (b, 0, 0)),
            pl.BlockSpec((bs, cu, tn), lambda b, t: (b, 0, t)),
            pl.BlockSpec((bs, ck, m), lambda b, t: (b, 0, 0)),
            pl.BlockSpec((o, cin), lambda b, t: (0, 0)),
            pl.BlockSpec((o, 1), lambda b, t: (0, 0)),
            pl.BlockSpec((o, o), lambda b, t: (0, 0)),
            pl.BlockSpec((o, 1), lambda b, t: (0, 0)),
        ],
        out_specs=pl.BlockSpec((bs, o, tn), lambda b, t: (b, 0, t)),
        out_shape=jax.ShapeDtypeStruct((B, o, n), jnp.float32),
        interpret=interpret,
    )(uxyz, kxyzt, ufeat, kfeat, w0, b0.reshape(o, 1), w1, b1.reshape(o, 1))


def kernel(l_xyz_0, l_xyz_1, l_xyz_2, l_xyz_3, l_xyz_4,
           l_features_0, l_features_1, l_features_2, l_features_3, l_features_4,
           fp4_w0, fp4_b0, fp4_w1, fp4_b1,
           fp3_w0, fp3_b0, fp3_w1, fp3_b1,
           fp2_w0, fp2_b0, fp2_w1, fp2_b1,
           fp1_w0, fp1_b0, fp1_w1, fp1_b1):
    f3 = _fp_level(l_xyz_3, l_xyz_4, l_features_3, l_features_4,
                   fp4_w0, fp4_b0, fp4_w1, fp4_b1, tn=64, bs=8)
    f2 = _fp_level(l_xyz_2, l_xyz_3, l_features_2, f3,
                   fp3_w0, fp3_b0, fp3_w1, fp3_b1, tn=256, bs=8)
    f1 = _fp_level(l_xyz_1, l_xyz_2, l_features_1, f2,
                   fp2_w0, fp2_b0, fp2_w1, fp2_b1, tn=1024, bs=2)
    f0 = _fp_level(l_xyz_0, l_xyz_1, l_features_0, f1,
                   fp1_w0, fp1_b0, fp1_w1, fp1_b1, tn=2048, bs=1)
    return f0


# speedup vs baseline: 1.2121x; 1.1454x over previous
import functools

import jax
import jax.numpy as jnp
from jax import lax
from jax.experimental import pallas as pl

_NN = (((1,), (1,)), ((), ()))
_NT = (((1,), (0,)), ((), ()))


def _split_dot_nn(a, b):
    ah = a.astype(jnp.bfloat16).astype(jnp.float32)
    al = a - ah
    bh = b.astype(jnp.bfloat16).astype(jnp.float32)
    bl = b - bh
    out = lax.dot_general(a, bl, _NN, preferred_element_type=jnp.float32)
    out += lax.dot_general(al, bh, _NN, preferred_element_type=jnp.float32)
    out += lax.dot_general(ah, bh, _NN, preferred_element_type=jnp.float32)
    return out


def _fp_kernel(uxyz_ref, kxyzt_ref, ufeat_ref, kfeat_ref,
               w0_ref, b0_ref, w1_ref, b1_ref, out_ref, *, m, bs,
               exact_interp=True):
    if bs == 1:
        u = uxyz_ref[0]
        kt = kxyzt_ref[0]
        uf = ufeat_ref[0]
        kf = kfeat_ref[0]
    else:
        u = jnp.concatenate([uxyz_ref[i] for i in range(bs)], axis=0)
        kt = jnp.concatenate([kxyzt_ref[i] for i in range(bs)], axis=1)
        uf = jnp.concatenate([ufeat_ref[i] for i in range(bs)], axis=1)
        kf = jnp.concatenate([kfeat_ref[i] for i in range(bs)], axis=1)
    tn = u.shape[0]
    sub_m = m
    m = m * bs

    u0, u1, u2 = u[:, 0:1], u[:, 1:2], u[:, 2:3]
    k0, k1, k2 = kt[0:1, :], kt[1:2, :], kt[2:3, :]
    uu = u0 * u0 + u1 * u1 + u2 * u2
    kk = k0 * k0 + k1 * k1 + k2 * k2
    uk = lax.dot_general(u, kt, _NT, preferred_element_type=jnp.float32)
    d2 = (uu + kk) - 2.0 * uk
    if bs > 1:
        br = lax.broadcasted_iota(jnp.int32, (tn, m), 0) // (tn // bs)
        bc = lax.broadcasted_iota(jnp.int32, (tn, m), 1) // sub_m
        d2 = jnp.where(br == bc, d2, jnp.float32(jnp.inf))

    iota = lax.broadcasted_iota(jnp.int32, (tn, m), 1).astype(jnp.float32)
    cur = d2
    idxs, rs = [], []
    for j in range(3):
        mn = jnp.min(cur, axis=1, keepdims=True)
        idx = jnp.min(jnp.where(cur == mn, iota, jnp.float32(m)),
                      axis=1, keepdims=True)
        idxs.append(idx)
        rs.append(1.0 / (jnp.maximum(mn, 0.0) + 1e-8))
        if j < 2:
            cur = jnp.where(iota == idx, jnp.float32(jnp.inf), cur)
    inv = 1.0 / ((rs[0] + rs[1]) + rs[2])
    w0_, w1_, w2_ = rs[0] * inv, rs[1] * inv, rs[2] * inv
    wmat = jnp.where(iota == idxs[0], w0_,
                     jnp.where(iota == idxs[1], w1_,
                               jnp.where(iota == idxs[2], w2_, 0.0)))

    if exact_interp:
        interp = _split_dot_nn(kf, wmat)
    else:
        interp = lax.dot_general(kf, wmat, _NN,
                                 preferred_element_type=jnp.float32)

    x = jnp.concatenate([interp, uf], axis=0)
    h = lax.dot_general(w0_ref[:], x, _NT, preferred_element_type=jnp.float32)
    h = jnp.maximum(h + b0_ref[:], 0.0)
    o = lax.dot_general(w1_ref[:], h, _NT, preferred_element_type=jnp.float32)
    res = jnp.maximum(o + b1_ref[:], 0.0)
    if bs == 1:
        out_ref[0] = res
    else:
        sub_n = tn // bs
        for i in range(bs):
            out_ref[i] = res[:, i * sub_n:(i + 1) * sub_n]


def _fp_level(uxyz, kxyz, ufeat, kfeat, w0, b0, w1, b1, tn, bs=1,
              exact_interp=True, interpret=False):
    B, n, _ = uxyz.shape
    m = kxyz.shape[1]
    cu = ufeat.shape[1]
    ck = kfeat.shape[1]
    o, cin = w0.shape
    grid = (B // bs, n // tn)
    kxyzt = jnp.transpose(kxyz, (0, 2, 1))
    return pl.pallas_call(
        functools.partial(_fp_kernel, m=m, bs=bs, exact_interp=exact_interp),
        grid=grid,
        in_specs=[
            pl.BlockSpec((bs, tn, 3), lambda b, t: (b, t, 0)),
            pl.BlockSpec((bs, 3, m), lambda b, t: (b, 0, 0)),
            pl.BlockSpec((bs, cu, tn), lambda b, t: (b, 0, t)),
            pl.BlockSpec((bs, ck, m), lambda b, t: (b, 0, 0)),
            pl.BlockSpec((o, cin), lambda b, t: (0, 0)),
            pl.BlockSpec((o, 1), lambda b, t: (0, 0)),
            pl.BlockSpec((o, o), lambda b, t: (0, 0)),
            pl.BlockSpec((o, 1), lambda b, t: (0, 0)),
        ],
        out_specs=pl.BlockSpec((bs, o, tn), lambda b, t: (b, 0, t)),
        out_shape=jax.ShapeDtypeStruct((B, o, n), jnp.float32),
        interpret=interpret,
    )(uxyz, kxyzt, ufeat, kfeat, w0, b0.reshape(o, 1), w1, b1.reshape(o, 1))


def kernel(l_xyz_0, l_xyz_1, l_xyz_2, l_xyz_3, l_xyz_4,
           l_features_0, l_features_1, l_features_2, l_features_3, l_features_4,
           fp4_w0, fp4_b0, fp4_w1, fp4_b1,
           fp3_w0, fp3_b0, fp3_w1, fp3_b1,
           fp2_w0, fp2_b0, fp2_w1, fp2_b1,
           fp1_w0, fp1_b0, fp1_w1, fp1_b1):
    f3 = _fp_level(l_xyz_3, l_xyz_4, l_features_3, l_features_4,
                   fp4_w0, fp4_b0, fp4_w1, fp4_b1, tn=64, bs=8)
    f2 = _fp_level(l_xyz_2, l_xyz_3, l_features_2, f3,
                   fp3_w0, fp3_b0, fp3_w1, fp3_b1, tn=256, bs=8)
    f1 = _fp_level(l_xyz_1, l_xyz_2, l_features_1, f2,
                   fp2_w0, fp2_b0, fp2_w1, fp2_b1, tn=1024, bs=2)
    f0 = _fp_level(l_xyz_0, l_xyz_1, l_features_0, f1,
                   fp1_w0, fp1_b0, fp1_w1, fp1_b1, tn=2048, bs=1,
                   exact_interp=False)
    return f0


# speedup vs baseline: 1.2585x; 1.0383x over previous
import functools

import jax
import jax.numpy as jnp
from jax import lax
from jax.experimental import pallas as pl

_NN = (((1,), (1,)), ((), ()))
_NT = (((1,), (0,)), ((), ()))


def _split_dot_nn(a, b):
    ah = a.astype(jnp.bfloat16).astype(jnp.float32)
    al = a - ah
    bh = b.astype(jnp.bfloat16).astype(jnp.float32)
    bl = b - bh
    out = lax.dot_general(a, bl, _NN, preferred_element_type=jnp.float32)
    out += lax.dot_general(al, bh, _NN, preferred_element_type=jnp.float32)
    out += lax.dot_general(ah, bh, _NN, preferred_element_type=jnp.float32)
    return out


def _fp_kernel(uxyz_ref, kxyzt_ref, ufeat_ref, kfeat_ref,
               w0_ref, b0_ref, w1_ref, b1_ref, out_ref, *, m, bs,
               exact_interp=True):
    if bs == 1:
        u = uxyz_ref[0]
        kt = kxyzt_ref[0]
        uf = ufeat_ref[0]
        kf = kfeat_ref[0]
    else:
        u = jnp.concatenate([uxyz_ref[i] for i in range(bs)], axis=0)
        kt = jnp.concatenate([kxyzt_ref[i] for i in range(bs)], axis=1)
        uf = jnp.concatenate([ufeat_ref[i] for i in range(bs)], axis=1)
        kf = jnp.concatenate([kfeat_ref[i] for i in range(bs)], axis=1)
    tn = u.shape[0]
    sub_m = m
    m = m * bs

    u0, u1, u2 = u[:, 0:1], u[:, 1:2], u[:, 2:3]
    k0, k1, k2 = kt[0:1, :], kt[1:2, :], kt[2:3, :]
    uu = u0 * u0 + u1 * u1 + u2 * u2
    kk = k0 * k0 + k1 * k1 + k2 * k2
    uk = lax.dot_general(u, kt, _NT, preferred_element_type=jnp.float32)
    d2 = (uu + kk) - 2.0 * uk
    if bs > 1:
        br = lax.broadcasted_iota(jnp.int32, (tn, m), 0) // (tn // bs)
        bc = lax.broadcasted_iota(jnp.int32, (tn, m), 1) // sub_m
        d2 = jnp.where(br == bc, d2, jnp.float32(jnp.inf))

    iota = lax.broadcasted_iota(jnp.int32, (tn, m), 1).astype(jnp.float32)
    cur = d2
    idxs, rs = [], []
    for j in range(3):
        mn = jnp.min(cur, axis=1, keepdims=True)
        idx = jnp.min(jnp.where(cur == mn, iota, jnp.float32(m)),
                      axis=1, keepdims=True)
        idxs.append(idx)
        rs.append(1.0 / (jnp.maximum(mn, 0.0) + 1e-8))
        if j < 2:
            cur = jnp.where(iota == idx, jnp.float32(jnp.inf), cur)
    inv = 1.0 / ((rs[0] + rs[1]) + rs[2])
    w0_, w1_, w2_ = rs[0] * inv, rs[1] * inv, rs[2] * inv
    wmat = jnp.where(iota == idxs[0], w0_,
                     jnp.where(iota == idxs[1], w1_,
                               jnp.where(iota == idxs[2], w2_, 0.0)))

    if exact_interp:
        interp = _split_dot_nn(kf, wmat)
    else:
        interp = lax.dot_general(kf, wmat, _NN,
                                 preferred_element_type=jnp.float32)

    x = jnp.concatenate([interp, uf], axis=0)
    h = lax.dot_general(w0_ref[:], x, _NT, preferred_element_type=jnp.float32)
    h = jnp.maximum(h + b0_ref[:], 0.0)
    o = lax.dot_general(w1_ref[:], h, _NT, preferred_element_type=jnp.float32)
    res = jnp.maximum(o + b1_ref[:], 0.0)
    if bs == 1:
        out_ref[0] = res
    else:
        sub_n = tn // bs
        for i in range(bs):
            out_ref[i] = res[:, i * sub_n:(i + 1) * sub_n]


def _fp_level(uxyz, kxyz, ufeat, kfeat, w0, b0, w1, b1, tn, bs=1,
              exact_interp=True, interpret=False):
    B, n, _ = uxyz.shape
    m = kxyz.shape[1]
    cu = ufeat.shape[1]
    ck = kfeat.shape[1]
    o, cin = w0.shape
    grid = (B // bs, n // tn)
    kxyzt = jnp.transpose(kxyz, (0, 2, 1))
    return pl.pallas_call(
        functools.partial(_fp_kernel, m=m, bs=bs, exact_interp=exact_interp),
        grid=grid,
        in_specs=[
            pl.BlockSpec((bs, tn, 3), lambda b, t: (b, t, 0)),
            pl.BlockSpec((bs, 3, m), lambda b, t: (b, 0, 0)),
            pl.BlockSpec((bs, cu, tn), lambda b, t: (b, 0, t)),
            pl.BlockSpec((bs, ck, m), lambda b, t: (b, 0, 0)),
            pl.BlockSpec((o, cin), lambda b, t: (0, 0)),
            pl.BlockSpec((o, 1), lambda b, t: (0, 0)),
            pl.BlockSpec((o, o), lambda b, t: (0, 0)),
            pl.BlockSpec((o, 1), lambda b, t: (0, 0)),
        ],
        out_specs=pl.BlockSpec((bs, o, tn), lambda b, t: (b, 0, t)),
        out_shape=jax.ShapeDtypeStruct((B, o, n), jnp.float32),
        interpret=interpret,
    )(uxyz, kxyzt, ufeat, kfeat, w0, b0.reshape(o, 1), w1, b1.reshape(o, 1))


def kernel(l_xyz_0, l_xyz_1, l_xyz_2, l_xyz_3, l_xyz_4,
           l_features_0, l_features_1, l_features_2, l_features_3, l_features_4,
           fp4_w0, fp4_b0, fp4_w1, fp4_b1,
           fp3_w0, fp3_b0, fp3_w1, fp3_b1,
           fp2_w0, fp2_b0, fp2_w1, fp2_b1,
           fp1_w0, fp1_b0, fp1_w1, fp1_b1):
    f3 = _fp_level(l_xyz_3, l_xyz_4, l_features_3, l_features_4,
                   fp4_w0, fp4_b0, fp4_w1, fp4_b1, tn=64, bs=8)
    f2 = _fp_level(l_xyz_2, l_xyz_3, l_features_2, f3,
                   fp3_w0, fp3_b0, fp3_w1, fp3_b1, tn=256, bs=8)
    f1 = _fp_level(l_xyz_1, l_xyz_2, l_features_1, f2,
                   fp2_w0, fp2_b0, fp2_w1, fp2_b1, tn=1024, bs=2,
                   exact_interp=False)
    f0 = _fp_level(l_xyz_0, l_xyz_1, l_features_0, f1,
                   fp1_w0, fp1_b0, fp1_w1, fp1_b1, tn=2048, bs=1,
                   exact_interp=False)
    return f0
